# baseline (speedup 1.0000x reference)
import functools

import jax
import jax.numpy as jnp
from jax.experimental import pallas as pl
from jax.experimental.pallas import tpu as pltpu


def _gcn_body(x_ref, adj_ref, w1_ref, b1_ref, w2_ref, b2_ref,
              feat_ref, out_ref, support_ref, hacc_ref, *, n_k):
    i = pl.program_id(0)
    j = pl.program_id(1)

    @pl.when((i == 0) & (j == 0))
    def _compute_support():
        support_ref[...] = (
            jnp.dot(x_ref[...].astype(jnp.bfloat16),
                    w1_ref[...].astype(jnp.bfloat16),
                    preferred_element_type=jnp.float32)
            + b1_ref[...]
        ).astype(jnp.bfloat16)

    bk = adj_ref.shape[1]
    partial = jnp.dot(adj_ref[...].astype(jnp.bfloat16),
                      support_ref[pl.ds(j * bk, bk), :],
                      preferred_element_type=jnp.float32)

    @pl.when(j == 0)
    def _init():
        hacc_ref[...] = partial

    @pl.when(j > 0)
    def _accum():
        hacc_ref[...] += partial

    @pl.when(j == n_k - 1)
    def _finalize():
        feat = jnp.maximum(hacc_ref[...], 0.0)
        feat_ref[...] = feat
        out_ref[...] = jax.nn.sigmoid(
            jnp.dot(feat.astype(jnp.bfloat16), w2_ref[...].astype(jnp.bfloat16),
                    preferred_element_type=jnp.float32)
            + b2_ref[...]
        )


@functools.partial(jax.jit, static_argnames=("block_n", "block_k"))
def _gcn_fused(x, adj, W1, b1, W2, b2, block_n=1024, block_k=1024):
    n, f = x.shape
    h_dim = W1.shape[1]
    c = W2.shape[1]
    n_k = n // block_k
    b1r = b1.reshape(1, h_dim)
    b2r = b2.reshape(1, c)
    feature, out = pl.pallas_call(
        functools.partial(_gcn_body, n_k=n_k),
        grid=(n // block_n, n_k),
        in_specs=[
            pl.BlockSpec((n, f), lambda i, j: (0, 0)),
            pl.BlockSpec((block_n, block_k), lambda i, j: (i, j)),
            pl.BlockSpec((f, h_dim), lambda i, j: (0, 0)),
            pl.BlockSpec((1, h_dim), lambda i, j: (0, 0)),
            pl.BlockSpec((h_dim, c), lambda i, j: (0, 0)),
            pl.BlockSpec((1, c), lambda i, j: (0, 0)),
        ],
        out_specs=[
            pl.BlockSpec((block_n, h_dim), lambda i, j: (i, 0)),
            pl.BlockSpec((block_n, c), lambda i, j: (i, 0)),
        ],
        out_shape=[
            jax.ShapeDtypeStruct((n, h_dim), jnp.float32),
            jax.ShapeDtypeStruct((n, c), jnp.float32),
        ],
        scratch_shapes=[
            pltpu.VMEM((n, h_dim), jnp.bfloat16),
            pltpu.VMEM((block_n, h_dim), jnp.float32),
        ],
        compiler_params=pltpu.CompilerParams(
            dimension_semantics=("arbitrary", "arbitrary"),
        ),
    )(x, adj, W1, b1r, W2, b2r)
    return feature, out


def kernel(x, adj, W1, b1, W2, b2):
    return _gcn_fused(x, adj, W1, b1, W2, b2)


# speedup vs baseline: 1.1362x; 1.1362x over previous
import functools

import jax
import jax.numpy as jnp
from jax.experimental import pallas as pl
from jax.experimental.pallas import tpu as pltpu


def _gcn_body(x_ref, adj_ref, w1_ref, b1_ref, w2_ref, b2_ref,
              feat_ref, out_ref, support_ref, hacc_ref, *, n_k):
    i = pl.program_id(0)
    j = pl.program_id(1)

    @pl.when((i == 0) & (j == 0))
    def _compute_support():
        support_ref[...] = (
            jnp.dot(x_ref[...].astype(jnp.bfloat16),
                    w1_ref[...].astype(jnp.bfloat16),
                    preferred_element_type=jnp.float32)
            + b1_ref[...]
        ).astype(jnp.bfloat16)

    bk = adj_ref.shape[1]
    partial = jnp.dot(adj_ref[...].astype(jnp.bfloat16),
                      support_ref[pl.ds(j * bk, bk), :],
                      preferred_element_type=jnp.float32)

    @pl.when(j < n_k - 1)
    def _stash():
        @pl.when(j == 0)
        def _init():
            hacc_ref[...] = partial

        @pl.when(j > 0)
        def _accum():
            hacc_ref[...] += partial

    @pl.when(j == n_k - 1)
    def _finalize():
        feat = jnp.maximum(hacc_ref[...] + partial, 0.0)
        feat_ref[...] = feat
        out_ref[...] = jax.nn.sigmoid(
            jnp.dot(feat.astype(jnp.bfloat16), w2_ref[...].astype(jnp.bfloat16),
                    preferred_element_type=jnp.float32)
            + b2_ref[...]
        )


@functools.partial(jax.jit, static_argnames=("block_n", "block_k"))
def _gcn_fused(x, adj, W1, b1, W2, b2, block_n=1024, block_k=2048):
    n, f = x.shape
    h_dim = W1.shape[1]
    c = W2.shape[1]
    n_k = n // block_k
    b1r = b1.reshape(1, h_dim)
    b2r = b2.reshape(1, c)
    feature, out = pl.pallas_call(
        functools.partial(_gcn_body, n_k=n_k),
        grid=(n // block_n, n_k),
        in_specs=[
            pl.BlockSpec((n, f), lambda i, j: (0, 0)),
            pl.BlockSpec((block_n, block_k), lambda i, j: (i, j)),
            pl.BlockSpec((f, h_dim), lambda i, j: (0, 0)),
            pl.BlockSpec((1, h_dim), lambda i, j: (0, 0)),
            pl.BlockSpec((h_dim, c), lambda i, j: (0, 0)),
            pl.BlockSpec((1, c), lambda i, j: (0, 0)),
        ],
        out_specs=[
            pl.BlockSpec((block_n, h_dim), lambda i, j: (i, 0)),
            pl.BlockSpec((block_n, c), lambda i, j: (i, 0)),
        ],
        out_shape=[
            jax.ShapeDtypeStruct((n, h_dim), jnp.float32),
            jax.ShapeDtypeStruct((n, c), jnp.float32),
        ],
        scratch_shapes=[
            pltpu.VMEM((n, h_dim), jnp.bfloat16),
            pltpu.VMEM((block_n, h_dim), jnp.float32),
        ],
        compiler_params=pltpu.CompilerParams(
            dimension_semantics=("arbitrary", "arbitrary"),
        ),
    )(x, adj, W1, b1r, W2, b2r)
    return feature, out


def kernel(x, adj, W1, b1, W2, b2):
    return _gcn_fused(x, adj, W1, b1, W2, b2)
